# baseline (speedup 1.0000x reference)
import functools

import jax
import jax.numpy as jnp
from jax import lax
from jax.experimental import pallas as pl
from jax.experimental.pallas import tpu as pltpu
from jax.experimental.pallas import tpu_sc as plsc

_VT = 2048
_CR = 64
_K = 100
_SUB = 128


_PAD_KEY = -2**31


def _u32(x):
    return lax.bitcast_convert_type(x, jnp.uint32)


def _i32(x):
    return lax.bitcast_convert_type(x, jnp.int32)


def _mono_key(lt, col, v_real):
    u = _u32(lt)
    keyu = jnp.where((u >> 31) != 0, ~u, u | jnp.uint32(0x80000000))
    key = _i32(keyu ^ jnp.uint32(0x80000000))
    return jnp.where(col < v_real, key, jnp.int32(_PAD_KEY))


def _mid_i32(a, b):
    x = jnp.uint32(0x80000000)
    au = _u32(a) ^ x
    bu = _u32(b) ^ x
    mu = au + ((bu - au + jnp.uint32(1)) >> 1)
    return _i32(mu ^ x)


def _make_sc_gather(bsz, d):
    info = plsc.get_sparse_core_info()
    nc, ns = info.num_cores, info.num_subcores
    nw = nc * ns
    bpw = bsz // nw

    @functools.partial(
        pl.kernel,
        mesh=plsc.VectorSubcoreMesh(core_axis_name="c", subcore_axis_name="s"),
        out_type=jax.ShapeDtypeStruct((bsz, 128), jnp.float32),
        scratch_types=[
            pltpu.VMEM((bpw,), jnp.int32),
            pltpu.VMEM((bpw, 128), jnp.float32),
            pltpu.SemaphoreType.DMA,
        ],
    )
    def k(table_hbm, idx_hbm, out_hbm, idx_v, rows_v, sem):
        wid = lax.axis_index("s") * nc + lax.axis_index("c")
        base = wid * bpw
        pltpu.sync_copy(idx_hbm.at[pl.ds(base, bpw)], idx_v)
        pltpu.async_copy(table_hbm.at[idx_v], rows_v, sem).wait()
        pltpu.sync_copy(rows_v, out_hbm.at[pl.ds(base, bpw)])

    return k


def _gru_kernel(x_ref, h_ref, wx_ref, wh_ref, b_ref, g_ref, beta_ref,
                hn_ref, hln_ref):
    d = x_ref.shape[1]
    x = x_ref[...]
    h = h_ref[...]
    gx = jnp.dot(x, wx_ref[...], preferred_element_type=jnp.float32)
    gh = jnp.dot(h, wh_ref[...], preferred_element_type=jnp.float32)
    bb = b_ref[...]
    z = jax.nn.sigmoid(gx[:, :d] + gh[:, :d] + bb[:, :d])
    r = jax.nn.sigmoid(gx[:, d:2 * d] + gh[:, d:2 * d] + bb[:, d:2 * d])
    n = jnp.tanh(gx[:, 2 * d:] + r * gh[:, 2 * d:] + bb[:, 2 * d:])
    hn = (1.0 - z) * h + z * n
    mu = jnp.mean(hn, axis=-1, keepdims=True)
    var = jnp.mean((hn - mu) ** 2, axis=-1, keepdims=True)
    hln = (hn - mu) / jnp.sqrt(var + 1e-8) * g_ref[...] + beta_ref[...]
    hn_ref[...] = hn
    hln_ref[...] = hln


def _sel_kernel(hn_ref, emb_ref, bstar_ref, istar_ref, keys, maxk,
                *, nt, v_real, vpad):
    j = pl.program_id(1)
    cr = keys.shape[0]

    @pl.when(j < nt)
    def _matmul_phase():
        lt = jnp.dot(hn_ref[...], emb_ref[...],
                     preferred_element_type=jnp.float32)
        col = j * _VT + lax.broadcasted_iota(jnp.int32, (cr, _VT), 1)
        key = _mono_key(lt, col, v_real)
        keys[:, pl.ds(j * _VT, _VT)] = key
        tm = jnp.max(key.reshape(cr, _VT // _SUB, _SUB), axis=1)
        maxk[:, pl.ds(j * _SUB, _SUB)] = tm

    @pl.when(j == nt)
    def _bisect_phase():
        mk = maxk[...]

        def cnt_f(t):
            return jnp.sum((keys[...] > t).astype(jnp.int32),
                           axis=1, keepdims=True)

        a = jnp.full((cr, 1), _PAD_KEY, jnp.int32)
        b = jnp.full((cr, 1), 2**31 - 2, jnp.int32)

        def mbody(_, ab):
            a, b = ab
            mid = _mid_i32(a, b)
            p = jnp.sum((mk > mid).astype(jnp.int32), axis=1,
                        keepdims=True) >= _K
            live = a < b
            return (jnp.where(live & p, mid, a),
                    jnp.where(live & ~p, mid - 1, b))

        a, b = lax.fori_loop(0, 32, mbody, (a, b))
        t100 = a
        rmax = jnp.max(mk, axis=1, keepdims=True)

        a = t100
        b = jnp.maximum(rmax - 1, t100)

        def fcond(ab):
            return jnp.any(ab[0] < ab[1])

        def fbody(ab):
            a, b = ab
            mid = _mid_i32(a, b)
            p = cnt_f(mid) >= _K
            live = a < b
            return (jnp.where(live & p, mid, a),
                    jnp.where(live & ~p, mid - 1, b))

        a, _ = lax.while_loop(fcond, fbody, (a, b))
        bstar = a + 1
        n_gt = cnt_f(bstar)
        cnt_eq = jnp.sum((keys[...] == bstar).astype(jnp.int32),
                         axis=1, keepdims=True)
        r = _K - n_gt

        done = cnt_eq == r
        big = jnp.int32(vpad)
        a2 = jnp.where(done, big, 0)
        b2 = jnp.where(done, big, vpad - 1)

        def icond(ab):
            return jnp.any(ab[0] < ab[1])

        def ibody(ab):
            a2, b2 = ab
            mid = (a2 + b2) >> 1
            colg = lax.broadcasted_iota(jnp.int32, (cr, vpad), 1)
            sel = (keys[...] == bstar) & (colg <= mid)
            cl = jnp.sum(sel.astype(jnp.int32), axis=1, keepdims=True)
            p = cl >= r
            live = a2 < b2
            return (jnp.where(live & ~p, mid + 1, a2),
                    jnp.where(live & p, mid, b2))

        a2, _ = lax.while_loop(icond, ibody, (a2, b2))
        bstar_ref[...] = bstar
        istar_ref[...] = a2


def _acc_kernel(hn_ref, emb_ref, btab_ref, bstar_ref, istar_ref, out_ref,
                acc, *, nt, v_real):
    j = pl.program_id(0)
    bsz = hn_ref.shape[0]
    lt = jnp.dot(hn_ref[...], emb_ref[...], preferred_element_type=jnp.float32)
    col = j * _VT + lax.broadcasted_iota(jnp.int32, (bsz, _VT), 1)
    key = _mono_key(lt, col, v_real)
    bs = bstar_ref[...]
    mask = (key > bs) | ((key == bs) & (col <= istar_ref[...]))
    ml = jnp.where(mask, lt, 0.0)

    @pl.when(j == 0)
    def _():
        acc[...] = jnp.zeros_like(acc)

    acc[...] += jnp.dot(ml, btab_ref[...], preferred_element_type=jnp.float32)

    @pl.when(j == nt - 1)
    def _():
        out_ref[...] = acc[...]


def kernel(seqs, length, topk, T_emb_weight, S_emb_weight, item_emb_weight,
           Wx, Wh, b, ln_gamma, ln_beta, H0):
    bsz = seqs.shape[0]
    d, v = item_emb_weight.shape
    steps = 4

    vpad = ((v + _VT - 1) // _VT) * _VT
    nt = vpad // _VT
    nc = bsz // _CR
    emb_pad = jnp.pad(item_emb_weight, ((0, 0), (0, vpad - v)))
    iota_col = jnp.arange(vpad, dtype=jnp.float32)[:, None]
    btab = jnp.concatenate([
        jnp.pad(T_emb_weight.T, ((0, vpad - v), (0, 0))),
        jnp.pad(S_emb_weight.T, ((0, vpad - v), (0, 0))),
        jnp.pad(item_emb_weight.T, ((0, vpad - v), (0, 0))),
        iota_col,
        jnp.zeros((vpad, 256 - 3 * d - 1), jnp.float32),
    ], axis=1)
    b2 = b.reshape(1, -1)
    g2 = ln_gamma.reshape(1, -1)
    beta2 = ln_beta.reshape(1, -1)

    gru_call = pl.pallas_call(
        _gru_kernel,
        in_specs=[
            pl.BlockSpec((bsz, d), lambda: (0, 0)),
            pl.BlockSpec((bsz, d), lambda: (0, 0)),
            pl.BlockSpec((d, 3 * d), lambda: (0, 0)),
            pl.BlockSpec((d, 3 * d), lambda: (0, 0)),
            pl.BlockSpec((1, 3 * d), lambda: (0, 0)),
            pl.BlockSpec((1, d), lambda: (0, 0)),
            pl.BlockSpec((1, d), lambda: (0, 0)),
        ],
        out_specs=[
            pl.BlockSpec((bsz, d), lambda: (0, 0)),
            pl.BlockSpec((bsz, d), lambda: (0, 0)),
        ],
        out_shape=[
            jax.ShapeDtypeStruct((bsz, d), jnp.float32),
            jax.ShapeDtypeStruct((bsz, d), jnp.float32),
        ],
    )

    sel_call = pl.pallas_call(
        functools.partial(_sel_kernel, nt=nt, v_real=v, vpad=vpad),
        grid=(nc, nt + 1),
        in_specs=[
            pl.BlockSpec((_CR, d), lambda c, j: (c, 0)),
            pl.BlockSpec((d, _VT), lambda c, j: (0, jnp.minimum(j, nt - 1))),
        ],
        out_specs=[
            pl.BlockSpec((_CR, 1), lambda c, j: (c, 0)),
            pl.BlockSpec((_CR, 1), lambda c, j: (c, 0)),
        ],
        out_shape=[
            jax.ShapeDtypeStruct((bsz, 1), jnp.int32),
            jax.ShapeDtypeStruct((bsz, 1), jnp.int32),
        ],
        scratch_shapes=[
            pltpu.VMEM((_CR, vpad), jnp.int32),
            pltpu.VMEM((_CR, (vpad // _VT) * _SUB), jnp.int32),
        ],
    )

    acc_call = pl.pallas_call(
        functools.partial(_acc_kernel, nt=nt, v_real=v),
        grid=(nt,),
        in_specs=[
            pl.BlockSpec((bsz, d), lambda j: (0, 0)),
            pl.BlockSpec((d, _VT), lambda j: (0, j)),
            pl.BlockSpec((_VT, 256), lambda j: (j, 0)),
            pl.BlockSpec((bsz, 1), lambda j: (0, 0)),
            pl.BlockSpec((bsz, 1), lambda j: (0, 0)),
        ],
        out_specs=pl.BlockSpec((bsz, 256), lambda j: (0, 0)),
        out_shape=jax.ShapeDtypeStruct((bsz, 256), jnp.float32),
        scratch_shapes=[pltpu.VMEM((bsz, 256), jnp.float32)],
    )

    emb_rows128 = jnp.pad(emb_pad.T, ((0, 0), (0, 128 - d)))
    X = _make_sc_gather(bsz, d)(emb_rows128, seqs.astype(jnp.int32))[:, :d]
    H = H0
    t_list, s_list = [], []
    seq_parts = [seqs.astype(jnp.float32)]
    for _ in range(steps):
        hn, hln = gru_call(X, H, Wx, Wh, b2, g2, beta2)
        H = hln
        bstar, istar = sel_call(hn, emb_pad)
        res = acc_call(hn, emb_pad, btab, bstar, istar)
        t_list.append(res[:, None, 0:d])
        s_list.append(res[:, None, d:2 * d])
        X = res[:, 2 * d:3 * d]
        seq_parts.append(res[:, 3 * d])
    t_out = jnp.concatenate(t_list, axis=1)
    s_out = jnp.concatenate(s_list, axis=1)
    seq_out = jnp.concatenate(seq_parts, axis=0).reshape(bsz, -1)
    return (t_out, s_out, seq_out)


# speedup vs baseline: 1.0410x; 1.0410x over previous
import functools

import jax
import jax.numpy as jnp
from jax import lax
from jax.experimental import pallas as pl
from jax.experimental.pallas import tpu as pltpu
from jax.experimental.pallas import tpu_sc as plsc

_VT = 2048
_CR = 64
_K = 100
_SUB = 128


_PAD_KEY = -2**31


def _u32(x):
    return lax.bitcast_convert_type(x, jnp.uint32)


def _i32(x):
    return lax.bitcast_convert_type(x, jnp.int32)


def _f32_to_key(x):
    u = _u32(x)
    keyu = jnp.where((u >> 31) != 0, ~u, u | jnp.uint32(0x80000000))
    return _i32(keyu ^ jnp.uint32(0x80000000))


def _key_to_f32(k):
    uk = _u32(k)
    fb = jnp.where(k >= 0, uk, ~(uk ^ jnp.uint32(0x80000000)))
    return lax.bitcast_convert_type(fb, jnp.float32)


def _mid_i32(a, b):
    x = jnp.uint32(0x80000000)
    au = _u32(a) ^ x
    bu = _u32(b) ^ x
    mu = au + ((bu - au + jnp.uint32(1)) >> 1)
    return _i32(mu ^ x)


def _make_sc_gather(bsz, d):
    info = plsc.get_sparse_core_info()
    nc, ns = info.num_cores, info.num_subcores
    nw = nc * ns
    bpw = bsz // nw

    @functools.partial(
        pl.kernel,
        mesh=plsc.VectorSubcoreMesh(core_axis_name="c", subcore_axis_name="s"),
        out_type=jax.ShapeDtypeStruct((bsz, 128), jnp.float32),
        scratch_types=[
            pltpu.VMEM((bpw,), jnp.int32),
            pltpu.VMEM((bpw, 128), jnp.float32),
            pltpu.SemaphoreType.DMA,
        ],
    )
    def k(table_hbm, idx_hbm, out_hbm, idx_v, rows_v, sem):
        wid = lax.axis_index("s") * nc + lax.axis_index("c")
        base = wid * bpw
        pltpu.sync_copy(idx_hbm.at[pl.ds(base, bpw)], idx_v)
        pltpu.async_copy(table_hbm.at[idx_v], rows_v, sem).wait()
        pltpu.sync_copy(rows_v, out_hbm.at[pl.ds(base, bpw)])

    return k


def _gru_kernel(x_ref, h_ref, wx_ref, wh_ref, b_ref, g_ref, beta_ref,
                hn_ref, hln_ref):
    d = x_ref.shape[1]
    x = x_ref[...]
    h = h_ref[...]
    gx = jnp.dot(x, wx_ref[...], preferred_element_type=jnp.float32)
    gh = jnp.dot(h, wh_ref[...], preferred_element_type=jnp.float32)
    bb = b_ref[...]
    z = jax.nn.sigmoid(gx[:, :d] + gh[:, :d] + bb[:, :d])
    r = jax.nn.sigmoid(gx[:, d:2 * d] + gh[:, d:2 * d] + bb[:, d:2 * d])
    n = jnp.tanh(gx[:, 2 * d:] + r * gh[:, 2 * d:] + bb[:, 2 * d:])
    hn = (1.0 - z) * h + z * n
    mu = jnp.mean(hn, axis=-1, keepdims=True)
    var = jnp.mean((hn - mu) ** 2, axis=-1, keepdims=True)
    hln = (hn - mu) / jnp.sqrt(var + 1e-8) * g_ref[...] + beta_ref[...]
    hn_ref[...] = hn
    hln_ref[...] = hln


def _sel_kernel(hn_ref, emb_ref, bstar_ref, istar_ref, keys, maxk,
                *, nt, v_real, vpad):
    j = pl.program_id(1)
    cr = keys.shape[0]

    @pl.when(j < nt)
    def _matmul_phase():
        lt = jnp.dot(hn_ref[...], emb_ref[...],
                     preferred_element_type=jnp.float32)
        col = j * _VT + lax.broadcasted_iota(jnp.int32, (cr, _VT), 1)
        lt = jnp.where(col < v_real, lt, float("-inf"))
        keys[:, pl.ds(j * _VT, _VT)] = lt
        tm = jnp.max(lt.reshape(cr, _VT // _SUB, _SUB), axis=1)
        maxk[:, pl.ds(j * _SUB, _SUB)] = tm

    @pl.when(j == nt)
    def _bisect_phase():
        mk = maxk[...]

        def cnt_f(t):
            return jnp.sum((keys[...] > _key_to_f32(t)).astype(jnp.int32),
                           axis=1, keepdims=True)

        a = jnp.full((cr, 1), _PAD_KEY, jnp.int32)
        b = jnp.full((cr, 1), 2**31 - 2, jnp.int32)

        def mbody(_, ab):
            a, b = ab
            mid = _mid_i32(a, b)
            p = jnp.sum((mk > _key_to_f32(mid)).astype(jnp.int32), axis=1,
                        keepdims=True) >= _K
            live = a < b
            return (jnp.where(live & p, mid, a),
                    jnp.where(live & ~p, mid - 1, b))

        a, b = lax.fori_loop(0, 32, mbody, (a, b))
        t100 = a
        rmax = jnp.max(mk, axis=1, keepdims=True)

        a = t100
        b = jnp.maximum(_f32_to_key(rmax) - 1, t100)

        def fcond(ab):
            return jnp.any(ab[0] < ab[1])

        def fbody(ab):
            a, b = ab
            mid = _mid_i32(a, b)
            p = cnt_f(mid) >= _K
            live = a < b
            return (jnp.where(live & p, mid, a),
                    jnp.where(live & ~p, mid - 1, b))

        a, _ = lax.while_loop(fcond, fbody, (a, b))
        bstar = a + 1
        fb = _key_to_f32(bstar)
        n_gt = cnt_f(bstar)
        cnt_eq = jnp.sum((keys[...] == fb).astype(jnp.int32),
                         axis=1, keepdims=True)
        r = _K - n_gt

        done = cnt_eq == r
        big = jnp.int32(vpad)
        a2 = jnp.where(done, big, 0)
        b2 = jnp.where(done, big, vpad - 1)

        def icond(ab):
            return jnp.any(ab[0] < ab[1])

        def ibody(ab):
            a2, b2 = ab
            mid = (a2 + b2) >> 1
            colg = lax.broadcasted_iota(jnp.int32, (cr, vpad), 1)
            sel = (keys[...] == fb) & (colg <= mid)
            cl = jnp.sum(sel.astype(jnp.int32), axis=1, keepdims=True)
            p = cl >= r
            live = a2 < b2
            return (jnp.where(live & ~p, mid + 1, a2),
                    jnp.where(live & p, mid, b2))

        a2, _ = lax.while_loop(icond, ibody, (a2, b2))
        bstar_ref[...] = bstar
        istar_ref[...] = a2


def _acc_kernel(hn_ref, emb_ref, btab_ref, bstar_ref, istar_ref, out_ref,
                acc, *, nt, v_real):
    j = pl.program_id(0)
    bsz = hn_ref.shape[0]
    lt = jnp.dot(hn_ref[...], emb_ref[...], preferred_element_type=jnp.float32)
    col = j * _VT + lax.broadcasted_iota(jnp.int32, (bsz, _VT), 1)
    fb = _key_to_f32(bstar_ref[...])
    mask = (lt > fb) | ((lt == fb) & (col <= istar_ref[...]))
    ml = jnp.where(mask, lt, 0.0)

    @pl.when(j == 0)
    def _():
        acc[...] = jnp.zeros_like(acc)

    acc[...] += jnp.dot(ml, btab_ref[...], preferred_element_type=jnp.float32)

    @pl.when(j == nt - 1)
    def _():
        out_ref[...] = acc[...]


def kernel(seqs, length, topk, T_emb_weight, S_emb_weight, item_emb_weight,
           Wx, Wh, b, ln_gamma, ln_beta, H0):
    bsz = seqs.shape[0]
    d, v = item_emb_weight.shape
    steps = 4

    vpad = ((v + _VT - 1) // _VT) * _VT
    nt = vpad // _VT
    nc = bsz // _CR
    emb_pad = jnp.pad(item_emb_weight, ((0, 0), (0, vpad - v)))
    iota_col = jnp.arange(vpad, dtype=jnp.float32)[:, None]
    btab = jnp.concatenate([
        jnp.pad(T_emb_weight.T, ((0, vpad - v), (0, 0))),
        jnp.pad(S_emb_weight.T, ((0, vpad - v), (0, 0))),
        jnp.pad(item_emb_weight.T, ((0, vpad - v), (0, 0))),
        iota_col,
        jnp.zeros((vpad, 256 - 3 * d - 1), jnp.float32),
    ], axis=1)
    b2 = b.reshape(1, -1)
    g2 = ln_gamma.reshape(1, -1)
    beta2 = ln_beta.reshape(1, -1)

    gru_call = pl.pallas_call(
        _gru_kernel,
        in_specs=[
            pl.BlockSpec((bsz, d), lambda: (0, 0)),
            pl.BlockSpec((bsz, d), lambda: (0, 0)),
            pl.BlockSpec((d, 3 * d), lambda: (0, 0)),
            pl.BlockSpec((d, 3 * d), lambda: (0, 0)),
            pl.BlockSpec((1, 3 * d), lambda: (0, 0)),
            pl.BlockSpec((1, d), lambda: (0, 0)),
            pl.BlockSpec((1, d), lambda: (0, 0)),
        ],
        out_specs=[
            pl.BlockSpec((bsz, d), lambda: (0, 0)),
            pl.BlockSpec((bsz, d), lambda: (0, 0)),
        ],
        out_shape=[
            jax.ShapeDtypeStruct((bsz, d), jnp.float32),
            jax.ShapeDtypeStruct((bsz, d), jnp.float32),
        ],
    )

    sel_call = pl.pallas_call(
        functools.partial(_sel_kernel, nt=nt, v_real=v, vpad=vpad),
        grid=(nc, nt + 1),
        in_specs=[
            pl.BlockSpec((_CR, d), lambda c, j: (c, 0)),
            pl.BlockSpec((d, _VT), lambda c, j: (0, jnp.minimum(j, nt - 1))),
        ],
        out_specs=[
            pl.BlockSpec((_CR, 1), lambda c, j: (c, 0)),
            pl.BlockSpec((_CR, 1), lambda c, j: (c, 0)),
        ],
        out_shape=[
            jax.ShapeDtypeStruct((bsz, 1), jnp.int32),
            jax.ShapeDtypeStruct((bsz, 1), jnp.int32),
        ],
        scratch_shapes=[
            pltpu.VMEM((_CR, vpad), jnp.float32),
            pltpu.VMEM((_CR, (vpad // _VT) * _SUB), jnp.float32),
        ],
    )

    acc_call = pl.pallas_call(
        functools.partial(_acc_kernel, nt=nt, v_real=v),
        grid=(nt,),
        in_specs=[
            pl.BlockSpec((bsz, d), lambda j: (0, 0)),
            pl.BlockSpec((d, _VT), lambda j: (0, j)),
            pl.BlockSpec((_VT, 256), lambda j: (j, 0)),
            pl.BlockSpec((bsz, 1), lambda j: (0, 0)),
            pl.BlockSpec((bsz, 1), lambda j: (0, 0)),
        ],
        out_specs=pl.BlockSpec((bsz, 256), lambda j: (0, 0)),
        out_shape=jax.ShapeDtypeStruct((bsz, 256), jnp.float32),
        scratch_shapes=[pltpu.VMEM((bsz, 256), jnp.float32)],
    )

    emb_rows128 = jnp.pad(emb_pad.T, ((0, 0), (0, 128 - d)))
    X = _make_sc_gather(bsz, d)(emb_rows128, seqs.astype(jnp.int32))[:, :d]
    H = H0
    t_list, s_list = [], []
    seq_parts = [seqs.astype(jnp.float32)]
    for _ in range(steps):
        hn, hln = gru_call(X, H, Wx, Wh, b2, g2, beta2)
        H = hln
        bstar, istar = sel_call(hn, emb_pad)
        res = acc_call(hn, emb_pad, btab, bstar, istar)
        t_list.append(res[:, None, 0:d])
        s_list.append(res[:, None, d:2 * d])
        X = res[:, 2 * d:3 * d]
        seq_parts.append(res[:, 3 * d])
    t_out = jnp.concatenate(t_list, axis=1)
    s_out = jnp.concatenate(s_list, axis=1)
    seq_out = jnp.concatenate(seq_parts, axis=0).reshape(bsz, -1)
    return (t_out, s_out, seq_out)


# speedup vs baseline: 1.0439x; 1.0028x over previous
import functools

import jax
import jax.numpy as jnp
from jax import lax
from jax.experimental import pallas as pl
from jax.experimental.pallas import tpu as pltpu
from jax.experimental.pallas import tpu_sc as plsc

_VT = 2048
_CR = 64
_K = 100
_SUB = 128


_PAD_KEY = -2**31


def _u32(x):
    return lax.bitcast_convert_type(x, jnp.uint32)


def _i32(x):
    return lax.bitcast_convert_type(x, jnp.int32)


def _f32_to_key(x):
    u = _u32(x)
    keyu = jnp.where((u >> 31) != 0, ~u, u | jnp.uint32(0x80000000))
    return _i32(keyu ^ jnp.uint32(0x80000000))


def _key_to_f32(k):
    uk = _u32(k)
    fb = jnp.where(k >= 0, uk, ~(uk ^ jnp.uint32(0x80000000)))
    return lax.bitcast_convert_type(fb, jnp.float32)


def _mid_i32(a, b):
    x = jnp.uint32(0x80000000)
    au = _u32(a) ^ x
    bu = _u32(b) ^ x
    mu = au + ((bu - au + jnp.uint32(1)) >> 1)
    return _i32(mu ^ x)


def _make_sc_gather(bsz, d):
    info = plsc.get_sparse_core_info()
    nc, ns = info.num_cores, info.num_subcores
    nw = nc * ns
    bpw = bsz // nw

    @functools.partial(
        pl.kernel,
        mesh=plsc.VectorSubcoreMesh(core_axis_name="c", subcore_axis_name="s"),
        out_type=jax.ShapeDtypeStruct((bsz, 128), jnp.float32),
        scratch_types=[
            pltpu.VMEM((bpw,), jnp.int32),
            pltpu.VMEM((bpw, 128), jnp.float32),
            pltpu.SemaphoreType.DMA,
        ],
    )
    def k(table_hbm, idx_hbm, out_hbm, idx_v, rows_v, sem):
        wid = lax.axis_index("s") * nc + lax.axis_index("c")
        base = wid * bpw
        pltpu.sync_copy(idx_hbm.at[pl.ds(base, bpw)], idx_v)
        pltpu.async_copy(table_hbm.at[idx_v], rows_v, sem).wait()
        pltpu.sync_copy(rows_v, out_hbm.at[pl.ds(base, bpw)])

    return k


def _gru_kernel(x_ref, h_ref, wx_ref, wh_ref, b_ref, g_ref, beta_ref,
                hn_ref, hln_ref):
    d = x_ref.shape[1]
    x = x_ref[...]
    h = h_ref[...]
    gx = jnp.dot(x, wx_ref[...], preferred_element_type=jnp.float32)
    gh = jnp.dot(h, wh_ref[...], preferred_element_type=jnp.float32)
    bb = b_ref[...]
    z = jax.nn.sigmoid(gx[:, :d] + gh[:, :d] + bb[:, :d])
    r = jax.nn.sigmoid(gx[:, d:2 * d] + gh[:, d:2 * d] + bb[:, d:2 * d])
    n = jnp.tanh(gx[:, 2 * d:] + r * gh[:, 2 * d:] + bb[:, 2 * d:])
    hn = (1.0 - z) * h + z * n
    mu = jnp.mean(hn, axis=-1, keepdims=True)
    var = jnp.mean((hn - mu) ** 2, axis=-1, keepdims=True)
    hln = (hn - mu) / jnp.sqrt(var + 1e-8) * g_ref[...] + beta_ref[...]
    hn_ref[...] = hn
    hln_ref[...] = hln


def _sel_kernel(hn_ref, emb_ref, bstar_ref, istar_ref, keys, maxk,
                *, nt, v_real, vpad):
    j = pl.program_id(1)
    cr = keys.shape[0]

    @pl.when(j < nt - 1)
    def _matmul_phase():
        lt = jnp.dot(hn_ref[...], emb_ref[...],
                     preferred_element_type=jnp.float32)
        keys[:, pl.ds(j * _VT, _VT)] = lt
        tm = jnp.max(lt.reshape(cr, _VT // _SUB, _SUB), axis=1)
        maxk[:, pl.ds(j * _SUB, _SUB)] = tm

    @pl.when(j == nt - 1)
    def _matmul_phase_pad():
        lt = jnp.dot(hn_ref[...], emb_ref[...],
                     preferred_element_type=jnp.float32)
        col = j * _VT + lax.broadcasted_iota(jnp.int32, (cr, _VT), 1)
        lt = jnp.where(col < v_real, lt, float("-inf"))
        keys[:, pl.ds(j * _VT, _VT)] = lt
        tm = jnp.max(lt.reshape(cr, _VT // _SUB, _SUB), axis=1)
        maxk[:, pl.ds(j * _SUB, _SUB)] = tm

    @pl.when(j == nt)
    def _bisect_phase():
        mk = maxk[...]

        def cnt_f(t):
            return jnp.sum((keys[...] > _key_to_f32(t)).astype(jnp.int32),
                           axis=1, keepdims=True)

        a = jnp.full((cr, 1), _PAD_KEY, jnp.int32)
        b = jnp.full((cr, 1), 2**31 - 2, jnp.int32)

        def mbody(_, ab):
            a, b = ab
            mid = _mid_i32(a, b)
            p = jnp.sum((mk > _key_to_f32(mid)).astype(jnp.int32), axis=1,
                        keepdims=True) >= _K
            live = a < b
            return (jnp.where(live & p, mid, a),
                    jnp.where(live & ~p, mid - 1, b))

        a, b = lax.fori_loop(0, 32, mbody, (a, b))
        t100 = a
        rmax = jnp.max(mk, axis=1, keepdims=True)

        a = t100
        b = jnp.maximum(_f32_to_key(rmax) - 1, t100)

        def fcond(ab):
            return jnp.any(ab[0] < ab[1])

        def fbody(ab):
            a, b = ab
            mid = _mid_i32(a, b)
            p = cnt_f(mid) >= _K
            live = a < b
            return (jnp.where(live & p, mid, a),
                    jnp.where(live & ~p, mid - 1, b))

        a, _ = lax.while_loop(fcond, fbody, (a, b))
        bstar = a + 1
        fb = _key_to_f32(bstar)
        n_gt = cnt_f(bstar)
        cnt_eq = jnp.sum((keys[...] == fb).astype(jnp.int32),
                         axis=1, keepdims=True)
        r = _K - n_gt

        done = cnt_eq == r
        big = jnp.int32(vpad)
        a2 = jnp.where(done, big, 0)
        b2 = jnp.where(done, big, vpad - 1)

        def icond(ab):
            return jnp.any(ab[0] < ab[1])

        def ibody(ab):
            a2, b2 = ab
            mid = (a2 + b2) >> 1
            colg = lax.broadcasted_iota(jnp.int32, (cr, vpad), 1)
            sel = (keys[...] == fb) & (colg <= mid)
            cl = jnp.sum(sel.astype(jnp.int32), axis=1, keepdims=True)
            p = cl >= r
            live = a2 < b2
            return (jnp.where(live & ~p, mid + 1, a2),
                    jnp.where(live & p, mid, b2))

        a2, _ = lax.while_loop(icond, ibody, (a2, b2))
        bstar_ref[...] = bstar
        istar_ref[...] = a2


def _acc_kernel(hn_ref, emb_ref, btab_ref, bstar_ref, istar_ref, out_ref,
                acc, *, nt, v_real):
    j = pl.program_id(0)
    bsz = hn_ref.shape[0]
    lt = jnp.dot(hn_ref[...], emb_ref[...], preferred_element_type=jnp.float32)
    col = j * _VT + lax.broadcasted_iota(jnp.int32, (bsz, _VT), 1)
    fb = _key_to_f32(bstar_ref[...])
    mask = (lt > fb) | ((lt == fb) & (col <= istar_ref[...]))
    ml = jnp.where(mask, lt, 0.0)

    @pl.when(j == 0)
    def _():
        acc[...] = jnp.zeros_like(acc)

    acc[...] += jnp.dot(ml, btab_ref[...], preferred_element_type=jnp.float32)

    @pl.when(j == nt - 1)
    def _():
        out_ref[...] = acc[...]


def kernel(seqs, length, topk, T_emb_weight, S_emb_weight, item_emb_weight,
           Wx, Wh, b, ln_gamma, ln_beta, H0):
    bsz = seqs.shape[0]
    d, v = item_emb_weight.shape
    steps = 4

    vpad = ((v + _VT - 1) // _VT) * _VT
    nt = vpad // _VT
    nc = bsz // _CR
    emb_pad = jnp.pad(item_emb_weight, ((0, 0), (0, vpad - v)))
    iota_col = jnp.arange(vpad, dtype=jnp.float32)[:, None]
    btab = jnp.concatenate([
        jnp.pad(T_emb_weight.T, ((0, vpad - v), (0, 0))),
        jnp.pad(S_emb_weight.T, ((0, vpad - v), (0, 0))),
        jnp.pad(item_emb_weight.T, ((0, vpad - v), (0, 0))),
        iota_col,
        jnp.zeros((vpad, 256 - 3 * d - 1), jnp.float32),
    ], axis=1)
    b2 = b.reshape(1, -1)
    g2 = ln_gamma.reshape(1, -1)
    beta2 = ln_beta.reshape(1, -1)

    gru_call = pl.pallas_call(
        _gru_kernel,
        in_specs=[
            pl.BlockSpec((bsz, d), lambda: (0, 0)),
            pl.BlockSpec((bsz, d), lambda: (0, 0)),
            pl.BlockSpec((d, 3 * d), lambda: (0, 0)),
            pl.BlockSpec((d, 3 * d), lambda: (0, 0)),
            pl.BlockSpec((1, 3 * d), lambda: (0, 0)),
            pl.BlockSpec((1, d), lambda: (0, 0)),
            pl.BlockSpec((1, d), lambda: (0, 0)),
        ],
        out_specs=[
            pl.BlockSpec((bsz, d), lambda: (0, 0)),
            pl.BlockSpec((bsz, d), lambda: (0, 0)),
        ],
        out_shape=[
            jax.ShapeDtypeStruct((bsz, d), jnp.float32),
            jax.ShapeDtypeStruct((bsz, d), jnp.float32),
        ],
    )

    sel_call = pl.pallas_call(
        functools.partial(_sel_kernel, nt=nt, v_real=v, vpad=vpad),
        grid=(nc, nt + 1),
        in_specs=[
            pl.BlockSpec((_CR, d), lambda c, j: (c, 0)),
            pl.BlockSpec((d, _VT), lambda c, j: (0, jnp.minimum(j, nt - 1))),
        ],
        out_specs=[
            pl.BlockSpec((_CR, 1), lambda c, j: (c, 0)),
            pl.BlockSpec((_CR, 1), lambda c, j: (c, 0)),
        ],
        out_shape=[
            jax.ShapeDtypeStruct((bsz, 1), jnp.int32),
            jax.ShapeDtypeStruct((bsz, 1), jnp.int32),
        ],
        scratch_shapes=[
            pltpu.VMEM((_CR, vpad), jnp.float32),
            pltpu.VMEM((_CR, (vpad // _VT) * _SUB), jnp.float32),
        ],
    )

    acc_call = pl.pallas_call(
        functools.partial(_acc_kernel, nt=nt, v_real=v),
        grid=(nt,),
        in_specs=[
            pl.BlockSpec((bsz, d), lambda j: (0, 0)),
            pl.BlockSpec((d, _VT), lambda j: (0, j)),
            pl.BlockSpec((_VT, 256), lambda j: (j, 0)),
            pl.BlockSpec((bsz, 1), lambda j: (0, 0)),
            pl.BlockSpec((bsz, 1), lambda j: (0, 0)),
        ],
        out_specs=pl.BlockSpec((bsz, 256), lambda j: (0, 0)),
        out_shape=jax.ShapeDtypeStruct((bsz, 256), jnp.float32),
        scratch_shapes=[pltpu.VMEM((bsz, 256), jnp.float32)],
    )

    emb_rows128 = jnp.pad(emb_pad.T, ((0, 0), (0, 128 - d)))
    X = _make_sc_gather(bsz, d)(emb_rows128, seqs.astype(jnp.int32))[:, :d]
    H = H0
    t_list, s_list = [], []
    seq_parts = [seqs.astype(jnp.float32)]
    for _ in range(steps):
        hn, hln = gru_call(X, H, Wx, Wh, b2, g2, beta2)
        H = hln
        bstar, istar = sel_call(hn, emb_pad)
        res = acc_call(hn, emb_pad, btab, bstar, istar)
        t_list.append(res[:, None, 0:d])
        s_list.append(res[:, None, d:2 * d])
        X = res[:, 2 * d:3 * d]
        seq_parts.append(res[:, 3 * d])
    t_out = jnp.concatenate(t_list, axis=1)
    s_out = jnp.concatenate(s_list, axis=1)
    seq_out = jnp.concatenate(seq_parts, axis=0).reshape(bsz, -1)
    return (t_out, s_out, seq_out)
